# baseline (speedup 1.0000x reference)
import functools

import jax
import jax.numpy as jnp
from jax import lax
from jax.experimental import pallas as pl
from jax.experimental.pallas import tpu as pltpu
from jax.experimental.pallas import tpu_sc as plsc

N = 10000
E = 320000
F_IN = 128
H_DIM = 64
N_CLASSES = 64
ALPHA = 0.1
NITER = 10

NC = 2
NS = 16
COLH = N_CLASSES // NC
EPT = E // NS
CHUNK = 128
NCH = 159
NBUF = 3
EPT_PAD = NCH * CHUNK
NPAD = 10240
NPT = NPAD // NS


def _mlp_body(a_ref, w1_ref, w2_ref, o_ref):
    h = jnp.maximum(
        jnp.dot(a_ref[...], w1_ref[...], preferred_element_type=jnp.float32),
        0.0,
    )
    o_ref[...] = jnp.dot(h, w2_ref[...], preferred_element_type=jnp.float32)


def _local_logits(attr_matrix, W1, W2):
    blk = 400
    return pl.pallas_call(
        _mlp_body,
        grid=(N // blk,),
        in_specs=[
            pl.BlockSpec((blk, F_IN), lambda i: (i, 0)),
            pl.BlockSpec((F_IN, H_DIM), lambda i: (0, 0)),
            pl.BlockSpec((H_DIM, N_CLASSES), lambda i: (0, 0)),
        ],
        out_specs=pl.BlockSpec((blk, N_CLASSES), lambda i: (i, 0)),
        out_shape=jax.ShapeDtypeStruct((N, N_CLASSES), jnp.float32),
    )(attr_matrix, W1, W2)


def _propagate(logits2, srcp, dstp, wp):
    mesh = plsc.VectorSubcoreMesh(
        core_axis_name="c", subcore_axis_name="s", num_cores=NC, num_subcores=NS
    )

    def body(logits_hbm, src_hbm, dst_hbm, w_hbm, z_hbm,
             idx_v, dst_v, w_v, rows_v, msgs_v, agg_sh, z_sh,
             gsems, ssems):
        c = lax.axis_index("c")
        s = lax.axis_index("s")
        row0 = s * NPT
        NU = NPT // CHUNK

        pltpu.sync_copy(src_hbm.at[s], idx_v)
        pltpu.sync_copy(dst_hbm.at[s], dst_v)
        pltpu.sync_copy(w_hbm.at[s], w_v)
        pltpu.sync_copy(
            logits_hbm.at[c, pl.ds(row0, NPT), :], z_sh.at[pl.ds(row0, NPT), :]
        )
        for u in range(NU):
            rsl = pl.ds(row0 + u * CHUNK, CHUNK)
            pltpu.sync_copy(logits_hbm.at[c, rsl, :], rows_v.at[0])
            pltpu.sync_copy(rows_v.at[0], z_hbm.at[c, rsl, :])
        def zfill0(r, carry2):
            z16 = jnp.zeros((16,), jnp.float32)
            msgs_v[1, r, pl.ds(0, 16)] = z16
            msgs_v[1, r, pl.ds(16, 16)] = z16
            return carry2
        lax.fori_loop(0, CHUNK, zfill0, 0, unroll=8)
        for u in range(NU):
            pltpu.sync_copy(
                msgs_v.at[1], agg_sh.at[pl.ds(row0 + u * CHUNK, CHUNK), :]
            )

        def scale_rows(rbuf, mbuf, j):
            for g in range(8):
                wg = w_v[j, g, :]
                for k in range(16):
                    e = g * 16 + k
                    w_s = wg[k]
                    mbuf[e, pl.ds(0, 16)] = rbuf[e, pl.ds(0, 16)] * w_s
                    mbuf[e, pl.ds(16, 16)] = rbuf[e, pl.ds(16, 16)] * w_s

        def edge_round(j0, carry):
            for b in range(NBUF):
                j = j0 * NBUF + b
                rbuf = rows_v.at[b]
                mbuf = msgs_v.at[b]
                pltpu.make_async_copy(
                    z_sh.at[idx_v.at[j]], rbuf, gsems[b]
                ).wait()

                @pl.when(j >= NBUF)
                def _():
                    pltpu.make_async_copy(
                        mbuf, agg_sh.at[dst_v.at[j]], ssems[b]
                    ).wait()

                scale_rows(rbuf, mbuf, j)
                pltpu.async_copy(
                    mbuf, agg_sh.at[dst_v.at[j]], ssems[b], add=True
                )

                jn = j + NBUF

                @pl.when(jnp.logical_and(jn < NCH, jn % 2 == 0))
                def _():
                    pltpu.async_copy(z_sh.at[idx_v.at[jn]], rbuf, gsems[b])

                @pl.when(jnp.logical_and(jn < NCH, jn % 2 == 1))
                def _():
                    pltpu.async_copy(
                        z_hbm.at[c].at[idx_v.at[jn]], rbuf, gsems[b]
                    )
            return carry

        def one_iter(it, carry):
            for b in range(NBUF):
                if b % 2 == 0:
                    pltpu.async_copy(z_sh.at[idx_v.at[b]], rows_v.at[b], gsems[b])
                else:
                    pltpu.async_copy(
                        z_hbm.at[c].at[idx_v.at[b]], rows_v.at[b], gsems[b]
                    )
            lax.fori_loop(0, NCH // NBUF, edge_round, 0)
            for b in range(NBUF):
                pltpu.make_async_copy(
                    msgs_v.at[b], agg_sh.at[dst_v.at[NCH - NBUF + b]], ssems[b]
                ).wait()
            plsc.subcore_barrier()
            def zfill(r, carry2):
                z16 = jnp.zeros((16,), jnp.float32)
                msgs_v[1, r, pl.ds(0, 16)] = z16
                msgs_v[1, r, pl.ds(16, 16)] = z16
                return carry2

            lax.fori_loop(0, CHUNK, zfill, 0, unroll=8)
            for u in range(NU):
                rsl = pl.ds(row0 + u * CHUNK, CHUNK)
                pltpu.sync_copy(agg_sh.at[rsl, :], rows_v.at[0])
                pltpu.sync_copy(msgs_v.at[1], agg_sh.at[rsl, :])
                pltpu.sync_copy(logits_hbm.at[c, rsl, :], rows_v.at[1])

                def blend(r, carry2):
                    for h in range(2):
                        sl = pl.ds(h * 16, 16)
                        msgs_v[0, r, sl] = (
                            (1.0 - ALPHA) * rows_v[0, r, sl]
                            + ALPHA * rows_v[1, r, sl]
                        )
                    return carry2

                lax.fori_loop(0, CHUNK, blend, 0, unroll=4)
                pltpu.sync_copy(msgs_v.at[0], z_sh.at[rsl, :])
                pltpu.sync_copy(msgs_v.at[0], z_hbm.at[c, rsl, :])

            plsc.subcore_barrier()
            return carry

        plsc.subcore_barrier()
        lax.fori_loop(0, NITER, one_iter, 0)

    fn = pl.kernel(
        body,
        out_type=jax.ShapeDtypeStruct((NC, NPAD, COLH), jnp.float32),
        mesh=mesh,
        compiler_params=pltpu.CompilerParams(use_tc_tiling_on_sc=False),
        scratch_types=[
            pltpu.VMEM((NCH, CHUNK), jnp.int32),
            pltpu.VMEM((NCH, CHUNK), jnp.int32),
            pltpu.VMEM((NCH, 8, 16), jnp.float32),
            pltpu.VMEM((NBUF, CHUNK, COLH), jnp.float32),
            pltpu.VMEM((NBUF, CHUNK, COLH), jnp.float32),
            pltpu.VMEM_SHARED((NPAD, COLH), jnp.float32),
            pltpu.VMEM_SHARED((NPAD, COLH), jnp.float32),
            [pltpu.SemaphoreType.DMA] * NBUF,
            [pltpu.SemaphoreType.DMA] * NBUF,
        ],
    )
    return fn(logits2, srcp, dstp, wp)


def kernel(attr_matrix, edge_index, edge_weight, W1, W2):
    logits = _local_logits(attr_matrix, W1, W2)
    logits_p = jnp.pad(logits, ((0, NPAD - N), (0, 0)))
    logits2 = jnp.stack([logits_p[:, :COLH], logits_p[:, COLH:]], axis=0)

    src = edge_index[0].reshape(NS, EPT)
    dst = edge_index[1].reshape(NS, EPT)
    w = edge_weight.reshape(NS, EPT)
    pad = EPT_PAD - EPT
    srcp = jnp.pad(src, ((0, 0), (0, pad))).reshape(NS, NCH, CHUNK)
    dstp = jnp.pad(dst, ((0, 0), (0, pad))).reshape(NS, NCH, CHUNK)
    wp = jnp.pad(w, ((0, 0), (0, pad))).reshape(NS, NCH, 8, 16)

    z2 = _propagate(logits2, srcp, dstp, wp)
    return jnp.concatenate([z2[0, :N], z2[1, :N]], axis=1)


# speedup vs baseline: 1.2636x; 1.2636x over previous
import functools

import jax
import jax.numpy as jnp
from jax import lax
from jax.experimental import pallas as pl
from jax.experimental.pallas import tpu as pltpu
from jax.experimental.pallas import tpu_sc as plsc

N = 10000
E = 320000
F_IN = 128
H_DIM = 64
N_CLASSES = 64
ALPHA = 0.1
NITER = 10

NC = 2
NS = 16
COLH = N_CLASSES // NC
EPT = E // NS
CHUNK = 128
NCH = 159
NBUF = 3
EPT_PAD = NCH * CHUNK
NPAD = 10240
NPT = NPAD // NS


def _mlp_body(a_ref, w1_ref, w2_ref, o_ref):
    h = jnp.maximum(
        jnp.dot(a_ref[...], w1_ref[...], preferred_element_type=jnp.float32),
        0.0,
    )
    o_ref[...] = jnp.dot(h, w2_ref[...], preferred_element_type=jnp.float32)


def _local_logits(attr_matrix, W1, W2):
    blk = 400
    return pl.pallas_call(
        _mlp_body,
        grid=(N // blk,),
        in_specs=[
            pl.BlockSpec((blk, F_IN), lambda i: (i, 0)),
            pl.BlockSpec((F_IN, H_DIM), lambda i: (0, 0)),
            pl.BlockSpec((H_DIM, N_CLASSES), lambda i: (0, 0)),
        ],
        out_specs=pl.BlockSpec((blk, N_CLASSES), lambda i: (i, 0)),
        out_shape=jax.ShapeDtypeStruct((N, N_CLASSES), jnp.float32),
    )(attr_matrix, W1, W2)


def _propagate(logits2, srcp, dstp, wp):
    mesh = plsc.VectorSubcoreMesh(
        core_axis_name="c", subcore_axis_name="s", num_cores=NC, num_subcores=NS
    )

    def body(logits_hbm, src_hbm, dst_hbm, w_hbm, z_hbm,
             idx_v, dst_v, w_v, rows_v, msgs_v, logslice_v, agg_sh, z_sh,
             gsems, ssems):
        c = lax.axis_index("c")
        s = lax.axis_index("s")
        row0 = s * NPT
        NU = NPT // CHUNK

        pltpu.sync_copy(src_hbm.at[s], idx_v)
        pltpu.sync_copy(dst_hbm.at[s], dst_v)
        pltpu.sync_copy(w_hbm.at[s], w_v)
        for u in range(NU):
            rsl = pl.ds(row0 + u * CHUNK, CHUNK)
            pltpu.sync_copy(logits_hbm.at[c, rsl, :], logslice_v)

            def pack_init(r, carry2):
                lo = logslice_v[r, pl.ds(0, 16)]
                hi = logslice_v[r, pl.ds(16, 16)]
                rows_v[0, r, :] = plsc.pack(
                    lo, hi, format=plsc.PackFormat.INTERLEAVED
                )
                return carry2

            lax.fori_loop(0, CHUNK, pack_init, 0, unroll=4)
            pltpu.sync_copy(rows_v.at[0], z_sh.at[rsl, :])
        def zfill0(r, carry2):
            z16 = jnp.zeros((16,), jnp.float32)
            msgs_v[1, r, pl.ds(0, 16)] = z16
            msgs_v[1, r, pl.ds(16, 16)] = z16
            return carry2
        lax.fori_loop(0, CHUNK, zfill0, 0, unroll=8)
        for u in range(NU):
            pltpu.sync_copy(
                msgs_v.at[1], agg_sh.at[pl.ds(row0 + u * CHUNK, CHUNK), :]
            )

        def scale_rows(rbuf, mbuf, j):
            for g in range(8):
                wg = w_v[j, g, :]
                for k in range(16):
                    e = g * 16 + k
                    w_s = wg[k]
                    lo, hi = plsc.unpack(
                        rbuf[e, :], format=plsc.PackFormat.INTERLEAVED
                    )
                    mbuf[e, pl.ds(0, 16)] = lo * w_s
                    mbuf[e, pl.ds(16, 16)] = hi * w_s

        def edge_round(j0, carry):
            for b in range(NBUF):
                j = j0 * NBUF + b
                rbuf = rows_v.at[b]
                mbuf = msgs_v.at[b]
                pltpu.make_async_copy(
                    z_sh.at[idx_v.at[j]], rbuf, gsems[b]
                ).wait()

                @pl.when(j >= NBUF)
                def _():
                    pltpu.make_async_copy(
                        mbuf, agg_sh.at[dst_v.at[j]], ssems[b]
                    ).wait()

                scale_rows(rbuf, mbuf, j)
                pltpu.async_copy(
                    mbuf, agg_sh.at[dst_v.at[j]], ssems[b], add=True
                )

                @pl.when(j < NCH - NBUF)
                def _():
                    pltpu.async_copy(
                        z_sh.at[idx_v.at[j + NBUF]], rbuf, gsems[b]
                    )
            return carry

        def one_iter(it, carry):
            for b in range(NBUF):
                pltpu.async_copy(
                    z_sh.at[idx_v.at[b]], rows_v.at[b], gsems[b]
                )
            lax.fori_loop(0, NCH // NBUF, edge_round, 0)
            for b in range(NBUF):
                pltpu.make_async_copy(
                    msgs_v.at[b], agg_sh.at[dst_v.at[NCH - NBUF + b]], ssems[b]
                ).wait()
            plsc.subcore_barrier()
            def zfill(r, carry2):
                z16 = jnp.zeros((16,), jnp.float32)
                msgs_v[1, r, pl.ds(0, 16)] = z16
                msgs_v[1, r, pl.ds(16, 16)] = z16
                return carry2

            lax.fori_loop(0, CHUNK, zfill, 0, unroll=8)
            for u in range(NU):
                rsl = pl.ds(row0 + u * CHUNK, CHUNK)
                pltpu.sync_copy(agg_sh.at[rsl, :], msgs_v.at[2])
                pltpu.sync_copy(msgs_v.at[1], agg_sh.at[rsl, :])
                pltpu.sync_copy(logits_hbm.at[c, rsl, :], logslice_v)

                def blend(r, carry2):
                    lo = (
                        (1.0 - ALPHA) * msgs_v[2, r, pl.ds(0, 16)]
                        + ALPHA * logslice_v[r, pl.ds(0, 16)]
                    )
                    hi = (
                        (1.0 - ALPHA) * msgs_v[2, r, pl.ds(16, 16)]
                        + ALPHA * logslice_v[r, pl.ds(16, 16)]
                    )
                    msgs_v[0, r, pl.ds(0, 16)] = lo
                    msgs_v[0, r, pl.ds(16, 16)] = hi
                    rows_v[0, r, :] = plsc.pack(
                        lo, hi, format=plsc.PackFormat.INTERLEAVED
                    )
                    return carry2

                lax.fori_loop(0, CHUNK, blend, 0, unroll=4)
                pltpu.sync_copy(rows_v.at[0], z_sh.at[rsl, :])

                @pl.when(it == NITER - 1)
                def _():
                    pltpu.sync_copy(msgs_v.at[0], z_hbm.at[c, rsl, :])

            plsc.subcore_barrier()
            return carry

        plsc.subcore_barrier()
        lax.fori_loop(0, NITER, one_iter, 0)

    fn = pl.kernel(
        body,
        out_type=jax.ShapeDtypeStruct((NC, NPAD, COLH), jnp.float32),
        mesh=mesh,
        compiler_params=pltpu.CompilerParams(
            use_tc_tiling_on_sc=False, needs_layout_passes=False
        ),
        scratch_types=[
            pltpu.VMEM((NCH, CHUNK), jnp.int32),
            pltpu.VMEM((NCH, CHUNK), jnp.int32),
            pltpu.VMEM((NCH, 8, 16), jnp.float32),
            pltpu.VMEM((NBUF, CHUNK, COLH), jnp.bfloat16),
            pltpu.VMEM((NBUF, CHUNK, COLH), jnp.float32),
            pltpu.VMEM((CHUNK, COLH), jnp.float32),
            pltpu.VMEM_SHARED((NPAD, COLH), jnp.float32),
            pltpu.VMEM_SHARED((NPAD, COLH), jnp.bfloat16),
            [pltpu.SemaphoreType.DMA] * NBUF,
            [pltpu.SemaphoreType.DMA] * NBUF,
        ],
    )
    return fn(logits2, srcp, dstp, wp)


def kernel(attr_matrix, edge_index, edge_weight, W1, W2):
    logits = _local_logits(attr_matrix, W1, W2)
    logits_p = jnp.pad(logits, ((0, NPAD - N), (0, 0)))
    logits2 = jnp.stack([logits_p[:, :COLH], logits_p[:, COLH:]], axis=0)

    src = edge_index[0].reshape(NS, EPT)
    dst = edge_index[1].reshape(NS, EPT)
    w = edge_weight.reshape(NS, EPT)
    pad = EPT_PAD - EPT
    srcp = jnp.pad(src, ((0, 0), (0, pad))).reshape(NS, NCH, CHUNK)
    dstp = jnp.pad(dst, ((0, 0), (0, pad))).reshape(NS, NCH, CHUNK)
    wp = jnp.pad(w, ((0, 0), (0, pad))).reshape(NS, NCH, 8, 16)

    z2 = _propagate(logits2, srcp, dstp, wp)
    return jnp.concatenate([z2[0, :N], z2[1, :N]], axis=1)


# speedup vs baseline: 1.2935x; 1.0236x over previous
import functools

import jax
import jax.numpy as jnp
from jax import lax
from jax.experimental import pallas as pl
from jax.experimental.pallas import tpu as pltpu
from jax.experimental.pallas import tpu_sc as plsc

N = 10000
E = 320000
F_IN = 128
H_DIM = 64
N_CLASSES = 64
ALPHA = 0.1
NITER = 10

NC = 2
NS = 16
COLH = N_CLASSES // NC
EPT = E // NS
CHUNK = 128
NCH = 159
NBUF = 3
EPT_PAD = NCH * CHUNK
NPAD = 10240
NPT = NPAD // NS


def _mlp_body(a_ref, w1_ref, w2_ref, o_ref):
    h = jnp.maximum(
        jnp.dot(a_ref[...], w1_ref[...], preferred_element_type=jnp.float32),
        0.0,
    )
    o_ref[...] = jnp.dot(h, w2_ref[...], preferred_element_type=jnp.float32)


def _local_logits(attr_matrix, W1, W2):
    blk = 400
    return pl.pallas_call(
        _mlp_body,
        grid=(N // blk,),
        in_specs=[
            pl.BlockSpec((blk, F_IN), lambda i: (i, 0)),
            pl.BlockSpec((F_IN, H_DIM), lambda i: (0, 0)),
            pl.BlockSpec((H_DIM, N_CLASSES), lambda i: (0, 0)),
        ],
        out_specs=pl.BlockSpec((blk, N_CLASSES), lambda i: (i, 0)),
        out_shape=jax.ShapeDtypeStruct((N, N_CLASSES), jnp.float32),
    )(attr_matrix, W1, W2)


def _propagate(logits2, srcp, dstp, wp):
    mesh = plsc.VectorSubcoreMesh(
        core_axis_name="c", subcore_axis_name="s", num_cores=NC, num_subcores=NS
    )

    def body(logits_hbm, src_hbm, dst_hbm, w_hbm, z_hbm,
             idx_v, dst_v, w_v, rows_v, msgs_v, logslice_v, agg_sh, z_sh,
             gsems, ssems):
        c = lax.axis_index("c")
        s = lax.axis_index("s")
        row0 = s * NPT
        NU = NPT // CHUNK

        pltpu.sync_copy(src_hbm.at[s], idx_v)
        pltpu.sync_copy(dst_hbm.at[s], dst_v)
        pltpu.sync_copy(w_hbm.at[s], w_v)
        for u in range(NU):
            rsl = pl.ds(row0 + u * CHUNK, CHUNK)
            pltpu.sync_copy(logits_hbm.at[rsl, pl.ds(c * COLH, COLH)], logslice_v)

            def pack_init(r, carry2):
                lo = logslice_v[r, pl.ds(0, 16)]
                hi = logslice_v[r, pl.ds(16, 16)]
                rows_v[0, r, :] = plsc.pack(
                    lo, hi, format=plsc.PackFormat.INTERLEAVED
                )
                return carry2

            lax.fori_loop(0, CHUNK, pack_init, 0, unroll=4)
            pltpu.sync_copy(rows_v.at[0], z_sh.at[rsl, :])
        def zfill0(r, carry2):
            z16 = jnp.zeros((16,), jnp.float32)
            msgs_v[1, r, pl.ds(0, 16)] = z16
            msgs_v[1, r, pl.ds(16, 16)] = z16
            return carry2
        lax.fori_loop(0, CHUNK, zfill0, 0, unroll=8)
        for u in range(NU):
            pltpu.sync_copy(
                msgs_v.at[1], agg_sh.at[pl.ds(row0 + u * CHUNK, CHUNK), :]
            )

        def scale_rows(rbuf, mbuf, j):
            for g in range(8):
                wg = w_v[j, g, :]
                for k in range(16):
                    e = g * 16 + k
                    w_s = wg[k]
                    lo, hi = plsc.unpack(
                        rbuf[e, :], format=plsc.PackFormat.INTERLEAVED
                    )
                    mbuf[e, pl.ds(0, 16)] = lo * w_s
                    mbuf[e, pl.ds(16, 16)] = hi * w_s

        def edge_round(j0, carry):
            for b in range(NBUF):
                j = j0 * NBUF + b
                rbuf = rows_v.at[b]
                mbuf = msgs_v.at[b]
                pltpu.make_async_copy(
                    z_sh.at[idx_v.at[j]], rbuf, gsems[b]
                ).wait()

                @pl.when(j >= NBUF)
                def _():
                    pltpu.make_async_copy(
                        mbuf, agg_sh.at[dst_v.at[j]], ssems[b]
                    ).wait()

                scale_rows(rbuf, mbuf, j)
                pltpu.async_copy(
                    mbuf, agg_sh.at[dst_v.at[j]], ssems[b], add=True
                )

                @pl.when(j < NCH - NBUF)
                def _():
                    pltpu.async_copy(
                        z_sh.at[idx_v.at[j + NBUF]], rbuf, gsems[b]
                    )
            return carry

        def one_iter(it, carry):
            for b in range(NBUF):
                pltpu.async_copy(
                    z_sh.at[idx_v.at[b]], rows_v.at[b], gsems[b]
                )
            lax.fori_loop(0, NCH // NBUF, edge_round, 0)
            for b in range(NBUF):
                pltpu.make_async_copy(
                    msgs_v.at[b], agg_sh.at[dst_v.at[NCH - NBUF + b]], ssems[b]
                ).wait()
            plsc.subcore_barrier()
            def zfill(r, carry2):
                z16 = jnp.zeros((16,), jnp.float32)
                msgs_v[1, r, pl.ds(0, 16)] = z16
                msgs_v[1, r, pl.ds(16, 16)] = z16
                return carry2

            lax.fori_loop(0, CHUNK, zfill, 0, unroll=8)
            for u in range(NU):
                rsl = pl.ds(row0 + u * CHUNK, CHUNK)
                pltpu.sync_copy(agg_sh.at[rsl, :], msgs_v.at[2])
                pltpu.sync_copy(msgs_v.at[1], agg_sh.at[rsl, :])
                pltpu.sync_copy(logits_hbm.at[rsl, pl.ds(c * COLH, COLH)], logslice_v)

                def blend(r, carry2):
                    lo = (
                        (1.0 - ALPHA) * msgs_v[2, r, pl.ds(0, 16)]
                        + ALPHA * logslice_v[r, pl.ds(0, 16)]
                    )
                    hi = (
                        (1.0 - ALPHA) * msgs_v[2, r, pl.ds(16, 16)]
                        + ALPHA * logslice_v[r, pl.ds(16, 16)]
                    )
                    msgs_v[0, r, pl.ds(0, 16)] = lo
                    msgs_v[0, r, pl.ds(16, 16)] = hi
                    rows_v[0, r, :] = plsc.pack(
                        lo, hi, format=plsc.PackFormat.INTERLEAVED
                    )
                    return carry2

                lax.fori_loop(0, CHUNK, blend, 0, unroll=4)
                pltpu.sync_copy(rows_v.at[0], z_sh.at[rsl, :])

                @pl.when(it == NITER - 1)
                def _():
                    pltpu.sync_copy(
                        msgs_v.at[0], z_hbm.at[rsl, pl.ds(c * COLH, COLH)]
                    )

            plsc.subcore_barrier()
            return carry

        plsc.subcore_barrier()
        lax.fori_loop(0, NITER, one_iter, 0)

    fn = pl.kernel(
        body,
        out_type=jax.ShapeDtypeStruct((NPAD, N_CLASSES), jnp.float32),
        mesh=mesh,
        compiler_params=pltpu.CompilerParams(
            use_tc_tiling_on_sc=False, needs_layout_passes=False
        ),
        scratch_types=[
            pltpu.VMEM((NCH, CHUNK), jnp.int32),
            pltpu.VMEM((NCH, CHUNK), jnp.int32),
            pltpu.VMEM((NCH, 8, 16), jnp.float32),
            pltpu.VMEM((NBUF, CHUNK, COLH), jnp.bfloat16),
            pltpu.VMEM((NBUF, CHUNK, COLH), jnp.float32),
            pltpu.VMEM((CHUNK, COLH), jnp.float32),
            pltpu.VMEM_SHARED((NPAD, COLH), jnp.float32),
            pltpu.VMEM_SHARED((NPAD, COLH), jnp.bfloat16),
            [pltpu.SemaphoreType.DMA] * NBUF,
            [pltpu.SemaphoreType.DMA] * NBUF,
        ],
    )
    return fn(logits2, srcp, dstp, wp)


def kernel(attr_matrix, edge_index, edge_weight, W1, W2):
    logits = _local_logits(attr_matrix, W1, W2)
    logits_p = jnp.pad(logits, ((0, NPAD - N), (0, 0)))

    src = edge_index[0].reshape(NS, EPT)
    dst = edge_index[1].reshape(NS, EPT)
    w = edge_weight.reshape(NS, EPT)
    pad = EPT_PAD - EPT
    srcp = jnp.pad(src, ((0, 0), (0, pad))).reshape(NS, NCH, CHUNK)
    dstp = jnp.pad(dst, ((0, 0), (0, pad))).reshape(NS, NCH, CHUNK)
    wp = jnp.pad(w, ((0, 0), (0, pad))).reshape(NS, NCH, 8, 16)

    z = _propagate(logits_p, srcp, dstp, wp)
    return z[:N]


# speedup vs baseline: 1.3792x; 1.0663x over previous
import functools

import jax
import jax.numpy as jnp
from jax import lax
from jax.experimental import pallas as pl
from jax.experimental.pallas import tpu as pltpu
from jax.experimental.pallas import tpu_sc as plsc

N = 10000
E = 320000
F_IN = 128
H_DIM = 64
N_CLASSES = 64
ALPHA = 0.1
NITER = 10

NC = 2
NS = 16
COLH = N_CLASSES // NC
EPT = E // NS
CHUNK = 128
NCH = 159
NBUF = 3
EPT_PAD = NCH * CHUNK
NPAD = 10240
NPT = NPAD // NS


def _mlp_body(a_ref, w1_ref, w2_ref, o_ref):
    h = jnp.maximum(
        jnp.dot(a_ref[...], w1_ref[...], preferred_element_type=jnp.float32),
        0.0,
    )
    o_ref[...] = jnp.dot(h, w2_ref[...], preferred_element_type=jnp.float32)


def _local_logits(attr_matrix, W1, W2):
    blk = 400
    return pl.pallas_call(
        _mlp_body,
        grid=(N // blk,),
        in_specs=[
            pl.BlockSpec((blk, F_IN), lambda i: (i, 0)),
            pl.BlockSpec((F_IN, H_DIM), lambda i: (0, 0)),
            pl.BlockSpec((H_DIM, N_CLASSES), lambda i: (0, 0)),
        ],
        out_specs=pl.BlockSpec((blk, N_CLASSES), lambda i: (i, 0)),
        out_shape=jax.ShapeDtypeStruct((N, N_CLASSES), jnp.float32),
    )(attr_matrix, W1, W2)


def _propagate(logits2, srcp, dstp, wp):
    mesh = plsc.VectorSubcoreMesh(
        core_axis_name="c", subcore_axis_name="s", num_cores=NC, num_subcores=NS
    )

    def body(logits_hbm, src_hbm, dst_hbm, w_hbm, z_hbm,
             idx_v, dst_v, w_v, rows_v, msgs_v, lstage_v, ustage_v, agg_sh,
             z_sh, gsems, ssems):
        c = lax.axis_index("c")
        s = lax.axis_index("s")
        row0 = s * NPT
        NU = NPT // CHUNK

        pltpu.sync_copy(src_hbm.at[s], idx_v)
        pltpu.sync_copy(dst_hbm.at[s], dst_v)
        pltpu.sync_copy(w_hbm.at[s], w_v)
        for u in range(NU):
            rsl = pl.ds(row0 + u * CHUNK, CHUNK)
            pltpu.sync_copy(
                logits_hbm.at[rsl, pl.ds(c * COLH, COLH)], lstage_v.at[0]
            )

            def pack_init(r, carry2):
                lo = lstage_v[0, r, pl.ds(0, 16)]
                hi = lstage_v[0, r, pl.ds(16, 16)]
                rows_v[0, r, :] = plsc.pack(
                    lo, hi, format=plsc.PackFormat.INTERLEAVED
                )
                return carry2

            lax.fori_loop(0, CHUNK, pack_init, 0, unroll=4)
            pltpu.sync_copy(rows_v.at[0], z_sh.at[rsl, :])
        def zfill0(r, carry2):
            z16 = jnp.zeros((16,), jnp.float32)
            msgs_v[1, r, pl.ds(0, 16)] = z16
            msgs_v[1, r, pl.ds(16, 16)] = z16
            return carry2
        lax.fori_loop(0, CHUNK, zfill0, 0, unroll=8)
        for u in range(NU):
            pltpu.sync_copy(
                msgs_v.at[1], agg_sh.at[pl.ds(row0 + u * CHUNK, CHUNK), :]
            )

        def scale_rows(rbuf, mbuf, j):
            for g in range(8):
                wg = w_v[j, g, :]
                for k in range(16):
                    e = g * 16 + k
                    w_s = wg[k]
                    lo, hi = plsc.unpack(
                        rbuf[e, :], format=plsc.PackFormat.INTERLEAVED
                    )
                    mbuf[e, pl.ds(0, 16)] = lo * w_s
                    mbuf[e, pl.ds(16, 16)] = hi * w_s

        def edge_round(j0, carry):
            for b in range(NBUF):
                j = j0 * NBUF + b
                rbuf = rows_v.at[b]
                mbuf = msgs_v.at[b]
                pltpu.make_async_copy(
                    z_sh.at[idx_v.at[j]], rbuf, gsems[b]
                ).wait()

                @pl.when(j >= NBUF)
                def _():
                    pltpu.make_async_copy(
                        mbuf, agg_sh.at[dst_v.at[j]], ssems[b]
                    ).wait()

                scale_rows(rbuf, mbuf, j)
                pltpu.async_copy(
                    mbuf, agg_sh.at[dst_v.at[j]], ssems[b], add=True
                )

                @pl.when(j < NCH - NBUF)
                def _():
                    pltpu.async_copy(
                        z_sh.at[idx_v.at[j + NBUF]], rbuf, gsems[b]
                    )
            return carry

        def one_iter(it, carry):
            for b in range(NBUF):
                pltpu.async_copy(
                    z_sh.at[idx_v.at[b]], rows_v.at[b], gsems[b]
                )
            lax.fori_loop(0, NCH // NBUF, edge_round, 0)
            for b in range(NBUF):
                pltpu.make_async_copy(
                    msgs_v.at[b], agg_sh.at[dst_v.at[NCH - NBUF + b]], ssems[b]
                ).wait()
            plsc.subcore_barrier()
            def zfill(r, carry2):
                z16 = jnp.zeros((16,), jnp.float32)
                msgs_v[1, r, pl.ds(0, 16)] = z16
                msgs_v[1, r, pl.ds(16, 16)] = z16
                return carry2

            lax.fori_loop(0, CHUNK, zfill, 0, unroll=8)

            def usl(u):
                return pl.ds(row0 + u * CHUNK, CHUNK)

            pltpu.async_copy(agg_sh.at[usl(0), :], ustage_v.at[0], gsems[0])
            pltpu.async_copy(
                logits_hbm.at[usl(0), pl.ds(c * COLH, COLH)],
                lstage_v.at[0], ssems[0],
            )
            for u in range(NU):
                pb, nb = u % 2, (u + 1) % 2
                rsl = usl(u)
                if u + 1 < NU:
                    pltpu.async_copy(
                        agg_sh.at[usl(u + 1), :], ustage_v.at[nb], gsems[nb]
                    )
                    pltpu.async_copy(
                        logits_hbm.at[usl(u + 1), pl.ds(c * COLH, COLH)],
                        lstage_v.at[nb], ssems[nb],
                    )
                pltpu.make_async_copy(
                    agg_sh.at[rsl, :], ustage_v.at[pb], gsems[pb]
                ).wait()
                pltpu.make_async_copy(
                    logits_hbm.at[rsl, pl.ds(c * COLH, COLH)],
                    lstage_v.at[pb], ssems[pb],
                ).wait()
                pltpu.async_copy(msgs_v.at[1], agg_sh.at[rsl, :], ssems[2])

                def blend(r, carry2):
                    lo = (
                        (1.0 - ALPHA) * ustage_v[pb, r, pl.ds(0, 16)]
                        + ALPHA * lstage_v[pb, r, pl.ds(0, 16)]
                    )
                    hi = (
                        (1.0 - ALPHA) * ustage_v[pb, r, pl.ds(16, 16)]
                        + ALPHA * lstage_v[pb, r, pl.ds(16, 16)]
                    )
                    msgs_v[0, r, pl.ds(0, 16)] = lo
                    msgs_v[0, r, pl.ds(16, 16)] = hi
                    rows_v[0, r, :] = plsc.pack(
                        lo, hi, format=plsc.PackFormat.INTERLEAVED
                    )
                    return carry2

                lax.fori_loop(0, CHUNK, blend, 0, unroll=4)
                pltpu.sync_copy(rows_v.at[0], z_sh.at[rsl, :])

                @pl.when(it == NITER - 1)
                def _():
                    pltpu.sync_copy(
                        msgs_v.at[0], z_hbm.at[rsl, pl.ds(c * COLH, COLH)]
                    )
            for u in range(NU):
                pltpu.make_async_copy(
                    msgs_v.at[1], agg_sh.at[usl(u), :], ssems[2]
                ).wait()

            plsc.subcore_barrier()
            return carry

        plsc.subcore_barrier()
        lax.fori_loop(0, NITER, one_iter, 0)

    fn = pl.kernel(
        body,
        out_type=jax.ShapeDtypeStruct((NPAD, N_CLASSES), jnp.float32),
        mesh=mesh,
        compiler_params=pltpu.CompilerParams(
            use_tc_tiling_on_sc=False, needs_layout_passes=False
        ),
        scratch_types=[
            pltpu.VMEM((NCH, CHUNK), jnp.int32),
            pltpu.VMEM((NCH, CHUNK), jnp.int32),
            pltpu.VMEM((NCH, 8, 16), jnp.float32),
            pltpu.VMEM((NBUF, CHUNK, COLH), jnp.bfloat16),
            pltpu.VMEM((NBUF, CHUNK, COLH), jnp.float32),
            pltpu.VMEM((2, CHUNK, COLH), jnp.float32),
            pltpu.VMEM((2, CHUNK, COLH), jnp.float32),
            pltpu.VMEM_SHARED((NPAD, COLH), jnp.float32),
            pltpu.VMEM_SHARED((NPAD, COLH), jnp.bfloat16),
            [pltpu.SemaphoreType.DMA] * NBUF,
            [pltpu.SemaphoreType.DMA] * NBUF,
        ],
    )
    return fn(logits2, srcp, dstp, wp)


def kernel(attr_matrix, edge_index, edge_weight, W1, W2):
    logits = _local_logits(attr_matrix, W1, W2)
    logits_p = jnp.pad(logits, ((0, NPAD - N), (0, 0)))

    src = edge_index[0].reshape(NS, EPT)
    dst = edge_index[1].reshape(NS, EPT)
    w = edge_weight.reshape(NS, EPT)
    pad = EPT_PAD - EPT
    srcp = jnp.pad(src, ((0, 0), (0, pad))).reshape(NS, NCH, CHUNK)
    dstp = jnp.pad(dst, ((0, 0), (0, pad))).reshape(NS, NCH, CHUNK)
    wp = jnp.pad(w, ((0, 0), (0, pad))).reshape(NS, NCH, 8, 16)

    z = _propagate(logits_p, srcp, dstp, wp)
    return z[:N]


# speedup vs baseline: 1.3914x; 1.0088x over previous
import functools

import jax
import jax.numpy as jnp
from jax import lax
from jax.experimental import pallas as pl
from jax.experimental.pallas import tpu as pltpu
from jax.experimental.pallas import tpu_sc as plsc

N = 10000
E = 320000
F_IN = 128
H_DIM = 64
N_CLASSES = 64
ALPHA = 0.1
NITER = 10

NC = 2
NS = 16
COLH = N_CLASSES // NC
EPT = E // NS
CHUNK = 128
NCH = 159
NBUF = 3
EPT_PAD = NCH * CHUNK
NPAD = 10240
NPT = NPAD // NS


def _mlp_body(a_ref, w1_ref, w2_ref, o_ref):
    h = jnp.maximum(
        jnp.dot(a_ref[...], w1_ref[...], preferred_element_type=jnp.float32),
        0.0,
    )
    o_ref[...] = jnp.dot(h, w2_ref[...], preferred_element_type=jnp.float32)


def _local_logits(attr_matrix, W1, W2):
    blk = 400
    return pl.pallas_call(
        _mlp_body,
        grid=(N // blk,),
        in_specs=[
            pl.BlockSpec((blk, F_IN), lambda i: (i, 0)),
            pl.BlockSpec((F_IN, H_DIM), lambda i: (0, 0)),
            pl.BlockSpec((H_DIM, N_CLASSES), lambda i: (0, 0)),
        ],
        out_specs=pl.BlockSpec((blk, N_CLASSES), lambda i: (i, 0)),
        out_shape=jax.ShapeDtypeStruct((N, N_CLASSES), jnp.float32),
    )(attr_matrix, W1, W2)


def _propagate(logits2, srcp, dstp, wp):
    mesh = plsc.VectorSubcoreMesh(
        core_axis_name="c", subcore_axis_name="s", num_cores=NC, num_subcores=NS
    )

    def body(logits_hbm, src_hbm, dst_hbm, w_hbm, z_hbm,
             idx_v, dst_v, w_v, rows_v, msgs_v, lstage_v, ustage_v, agg_sh,
             z_sh, gsems, ssems):
        c = lax.axis_index("c")
        s = lax.axis_index("s")
        row0 = s * NPT
        NU = NPT // CHUNK

        pltpu.sync_copy(src_hbm.at[s], idx_v)
        pltpu.sync_copy(dst_hbm.at[s], dst_v)
        pltpu.sync_copy(w_hbm.at[s], w_v)
        for u in range(NU):
            rsl = pl.ds(row0 + u * CHUNK, CHUNK)
            pltpu.sync_copy(
                logits_hbm.at[rsl, pl.ds(c * COLH, COLH)], lstage_v.at[0]
            )

            def pack_init(r, carry2):
                lo = lstage_v[0, r, pl.ds(0, 16)]
                hi = lstage_v[0, r, pl.ds(16, 16)]
                rows_v[0, r, :] = plsc.pack(
                    lo, hi, format=plsc.PackFormat.INTERLEAVED
                )
                return carry2

            lax.fori_loop(0, CHUNK, pack_init, 0, unroll=4)
            pltpu.sync_copy(rows_v.at[0], z_sh.at[rsl, :])
        def zfill0(r, carry2):
            z16 = jnp.zeros((16,), jnp.float32)
            msgs_v[1, r, pl.ds(0, 16)] = z16
            msgs_v[1, r, pl.ds(16, 16)] = z16
            return carry2
        lax.fori_loop(0, CHUNK, zfill0, 0, unroll=8)
        for u in range(NU):
            pltpu.sync_copy(
                msgs_v.at[1], agg_sh.at[pl.ds(row0 + u * CHUNK, CHUNK), :]
            )

        def scale_rows(rbuf, mbuf, j):
            for g in range(8):
                wg = w_v[j, g, :]
                for k in range(16):
                    e = g * 16 + k
                    w_s = wg[k]
                    lo, hi = plsc.unpack(
                        rbuf[e, :], format=plsc.PackFormat.INTERLEAVED
                    )
                    mbuf[e, pl.ds(0, 16)] = lo * w_s
                    mbuf[e, pl.ds(16, 16)] = hi * w_s

        def edge_round(j0, carry):
            for b in range(NBUF):
                j = j0 * NBUF + b
                rbuf = rows_v.at[b]
                mbuf = msgs_v.at[b]
                pltpu.make_async_copy(
                    z_sh.at[idx_v.at[j]], rbuf, gsems[b]
                ).wait()

                @pl.when(j >= NBUF)
                def _():
                    pltpu.make_async_copy(
                        mbuf, agg_sh.at[dst_v.at[j]], ssems[b]
                    ).wait()

                scale_rows(rbuf, mbuf, j)
                pltpu.async_copy(
                    mbuf, agg_sh.at[dst_v.at[j]], ssems[b], add=True
                )

                @pl.when(j < NCH - NBUF)
                def _():
                    pltpu.async_copy(
                        z_sh.at[idx_v.at[j + NBUF]], rbuf, gsems[b]
                    )
            return carry

        def one_iter(it, carry):
            for b in range(NBUF):
                pltpu.async_copy(
                    z_sh.at[idx_v.at[b]], rows_v.at[b], gsems[b]
                )
            lax.fori_loop(0, NCH // NBUF, edge_round, 0)
            for b in range(NBUF):
                pltpu.make_async_copy(
                    msgs_v.at[b], agg_sh.at[dst_v.at[NCH - NBUF + b]], ssems[b]
                ).wait()
            plsc.subcore_barrier()
            def zfill(r, carry2):
                z16 = jnp.zeros((16,), jnp.float32)
                msgs_v[1, r, pl.ds(0, 16)] = z16
                msgs_v[1, r, pl.ds(16, 16)] = z16
                return carry2

            lax.fori_loop(0, CHUNK, zfill, 0, unroll=8)

            def usl(u):
                return pl.ds(row0 + u * CHUNK, CHUNK)

            pltpu.async_copy(agg_sh.at[usl(0), :], ustage_v.at[0], gsems[0])
            pltpu.async_copy(
                logits_hbm.at[usl(0), pl.ds(c * COLH, COLH)],
                lstage_v.at[0], ssems[0],
            )
            for u in range(NU):
                pb, nb = u % 2, (u + 1) % 2
                rsl = usl(u)
                if u + 1 < NU:
                    pltpu.async_copy(
                        agg_sh.at[usl(u + 1), :], ustage_v.at[nb], gsems[nb]
                    )
                    pltpu.async_copy(
                        logits_hbm.at[usl(u + 1), pl.ds(c * COLH, COLH)],
                        lstage_v.at[nb], ssems[nb],
                    )
                pltpu.make_async_copy(
                    agg_sh.at[rsl, :], ustage_v.at[pb], gsems[pb]
                ).wait()
                pltpu.make_async_copy(
                    logits_hbm.at[rsl, pl.ds(c * COLH, COLH)],
                    lstage_v.at[pb], ssems[pb],
                ).wait()
                pltpu.async_copy(msgs_v.at[1], agg_sh.at[rsl, :], ssems[2])

                def blend(r, carry2):
                    lo = (
                        (1.0 - ALPHA) * ustage_v[pb, r, pl.ds(0, 16)]
                        + ALPHA * lstage_v[pb, r, pl.ds(0, 16)]
                    )
                    hi = (
                        (1.0 - ALPHA) * ustage_v[pb, r, pl.ds(16, 16)]
                        + ALPHA * lstage_v[pb, r, pl.ds(16, 16)]
                    )
                    msgs_v[0, r, pl.ds(0, 16)] = lo
                    msgs_v[0, r, pl.ds(16, 16)] = hi
                    rows_v[pb, r, :] = plsc.pack(
                        lo, hi, format=plsc.PackFormat.INTERLEAVED
                    )
                    return carry2

                lax.fori_loop(0, CHUNK, blend, 0, unroll=4)
                if u >= 2:
                    pltpu.make_async_copy(
                        rows_v.at[pb], z_sh.at[usl(u - 2), :], gsems[2]
                    ).wait()
                pltpu.async_copy(rows_v.at[pb], z_sh.at[rsl, :], gsems[2])

                @pl.when(it == NITER - 1)
                def _():
                    pltpu.sync_copy(
                        msgs_v.at[0], z_hbm.at[rsl, pl.ds(c * COLH, COLH)]
                    )
            for u in range(NU):
                pltpu.make_async_copy(
                    msgs_v.at[1], agg_sh.at[usl(u), :], ssems[2]
                ).wait()
            for u in range(NU - 2, NU):
                pltpu.make_async_copy(
                    rows_v.at[u % 2], z_sh.at[usl(u), :], gsems[2]
                ).wait()

            plsc.subcore_barrier()
            return carry

        plsc.subcore_barrier()
        lax.fori_loop(0, NITER, one_iter, 0)

    fn = pl.kernel(
        body,
        out_type=jax.ShapeDtypeStruct((NPAD, N_CLASSES), jnp.float32),
        mesh=mesh,
        compiler_params=pltpu.CompilerParams(
            use_tc_tiling_on_sc=False, needs_layout_passes=False
        ),
        scratch_types=[
            pltpu.VMEM((NCH, CHUNK), jnp.int32),
            pltpu.VMEM((NCH, CHUNK), jnp.int32),
            pltpu.VMEM((NCH, 8, 16), jnp.float32),
            pltpu.VMEM((NBUF, CHUNK, COLH), jnp.bfloat16),
            pltpu.VMEM((NBUF, CHUNK, COLH), jnp.float32),
            pltpu.VMEM((2, CHUNK, COLH), jnp.float32),
            pltpu.VMEM((2, CHUNK, COLH), jnp.float32),
            pltpu.VMEM_SHARED((NPAD, COLH), jnp.float32),
            pltpu.VMEM_SHARED((NPAD, COLH), jnp.bfloat16),
            [pltpu.SemaphoreType.DMA] * NBUF,
            [pltpu.SemaphoreType.DMA] * NBUF,
        ],
    )
    return fn(logits2, srcp, dstp, wp)


def kernel(attr_matrix, edge_index, edge_weight, W1, W2):
    logits = _local_logits(attr_matrix, W1, W2)
    logits_p = jnp.pad(logits, ((0, NPAD - N), (0, 0)))

    src = edge_index[0].reshape(NS, EPT)
    dst = edge_index[1].reshape(NS, EPT)
    w = edge_weight.reshape(NS, EPT)
    pad = EPT_PAD - EPT
    srcp = jnp.pad(src, ((0, 0), (0, pad))).reshape(NS, NCH, CHUNK)
    dstp = jnp.pad(dst, ((0, 0), (0, pad))).reshape(NS, NCH, CHUNK)
    wp = jnp.pad(w, ((0, 0), (0, pad))).reshape(NS, NCH, 8, 16)

    z = _propagate(logits_p, srcp, dstp, wp)
    return z[:N]
